# baseline (speedup 1.0000x reference)
import functools

import jax
import jax.numpy as jnp
from jax import lax
from jax.experimental import pallas as pl
from jax.experimental.pallas import tpu as pltpu
from jax.experimental.pallas import tpu_sc as plsc

N_NODES = 10000
N_EDGES = 320000
IN_DIM = 128
HID = 16

NC = 2
NS = 16
NW = NC * NS

CHUNK = 128
DEPTH = 4
EPW = N_EDGES // NW
CPW = 80
SLAB_C = CPW + DEPTH
E_PAD = NW * SLAB_C * CHUNK

ACC_ROWS = 10240
STRIP = ACC_ROWS // NS
DUMMY = N_NODES

_mesh = plsc.VectorSubcoreMesh(core_axis_name="c", subcore_axis_name="s")


@functools.partial(
    pl.kernel,
    out_type=jax.ShapeDtypeStruct((NC, ACC_ROWS), jnp.float32),
    mesh=_mesh,
    scratch_types=[
        pltpu.VMEM((SLAB_C, CHUNK), jnp.int32),
        pltpu.VMEM((CHUNK,), jnp.float32),
        pltpu.VMEM((STRIP,), jnp.float32),
        pltpu.VMEM_SHARED((ACC_ROWS,), jnp.float32),
    ] + [pltpu.SemaphoreType.DMA] * DEPTH,
    compiler_params=pltpu.CompilerParams(use_tc_tiling_on_sc=False),
)
def _sc_counts(dst_hbm, out_hbm, dst_v, ones_v, stage_v, acc_sh, *sems):
    cid = lax.axis_index("c")
    sid = lax.axis_index("s")
    wid = sid * NC + cid

    ones16 = jnp.ones((16,), jnp.float32)
    for i in range(CHUNK // 16):
        ones_v[pl.ds(i * 16, 16)] = ones16
    zero16 = jnp.zeros((16,), jnp.float32)

    def zero_body(i, carry):
        stage_v[pl.ds(i * 16, 16)] = zero16
        return carry

    lax.fori_loop(0, STRIP // 16, zero_body, 0)
    pltpu.sync_copy(stage_v, acc_sh.at[pl.ds(sid * STRIP, STRIP)])
    pltpu.sync_copy(dst_hbm.at[wid], dst_v)
    plsc.subcore_barrier()

    for b in range(DEPTH):
        pltpu.async_copy(ones_v, acc_sh.at[dst_v.at[b]], sems[b], add=True)

    def body(g, carry):
        for b in range(DEPTH):
            c = g * DEPTH + b
            pltpu.make_async_copy(ones_v, acc_sh.at[dst_v.at[0]], sems[b]).wait()
            pltpu.async_copy(ones_v, acc_sh.at[dst_v.at[c]], sems[b], add=True)
        return carry

    lax.fori_loop(1, CPW // DEPTH, body, 0)
    for b in range(DEPTH):
        pltpu.make_async_copy(ones_v, acc_sh.at[dst_v.at[0]], sems[b]).wait()
    plsc.subcore_barrier()
    pltpu.sync_copy(
        acc_sh.at[pl.ds(sid * STRIP, STRIP)],
        out_hbm.at[cid, pl.ds(sid * STRIP, STRIP)],
    )


@functools.partial(
    pl.kernel,
    out_type=jax.ShapeDtypeStruct((NC, ACC_ROWS, HID), jnp.float32),
    mesh=_mesh,
    scratch_types=[
        pltpu.VMEM((SLAB_C, CHUNK), jnp.int32),
        pltpu.VMEM((SLAB_C, CHUNK), jnp.int32),
    ] + [pltpu.VMEM((CHUNK, HID), jnp.float32)] * DEPTH
      + [pltpu.VMEM_SHARED((ACC_ROWS, HID), jnp.float32)]
      + [pltpu.SemaphoreType.DMA] * DEPTH,
    compiler_params=pltpu.CompilerParams(use_tc_tiling_on_sc=False),
)
def _sc_scatter_rows(vals_hbm, src_hbm, dst_hbm, out_hbm,
                     src_v, dst_v, *bufs_and_sems):
    bufs = bufs_and_sems[:DEPTH]
    acc_sh = bufs_and_sems[DEPTH]
    sems = bufs_and_sems[DEPTH + 1:]
    cid = lax.axis_index("c")
    sid = lax.axis_index("s")
    wid = sid * NC + cid

    zero16 = jnp.zeros((16,), jnp.float32)

    def zero_buf(i, carry):
        bufs[0][i, :] = zero16
        return carry

    lax.fori_loop(0, CHUNK, zero_buf, 0)
    for i in range(STRIP // CHUNK):
        pltpu.sync_copy(
            bufs[0], acc_sh.at[pl.ds(sid * STRIP + i * CHUNK, CHUNK)]
        )
    pltpu.sync_copy(src_hbm.at[wid], src_v)
    pltpu.sync_copy(dst_hbm.at[wid], dst_v)
    plsc.subcore_barrier()

    for b in range(DEPTH):
        pltpu.async_copy(vals_hbm.at[src_v.at[b]], bufs[b], sems[b])

    def body(g, carry):
        for b in range(DEPTH):
            c = g * DEPTH + b
            pltpu.make_async_copy(vals_hbm.at[src_v.at[0]], bufs[b], sems[b]).wait()
            pltpu.sync_copy(bufs[b], acc_sh.at[dst_v.at[c]], add=True)
            pltpu.async_copy(vals_hbm.at[src_v.at[c + DEPTH]], bufs[b], sems[b])
        return carry

    lax.fori_loop(0, CPW // DEPTH, body, 0)
    for b in range(DEPTH):
        pltpu.make_async_copy(vals_hbm.at[src_v.at[0]], bufs[b], sems[b]).wait()
    plsc.subcore_barrier()
    pltpu.sync_copy(
        acc_sh.at[pl.ds(sid * STRIP, STRIP)],
        out_hbm.at[cid, pl.ds(sid * STRIP, STRIP)],
    )


def _tc1_body(deg_ref, x_ref, w1_ref, g1_ref):
    dinv = lax.rsqrt(deg_ref[...])
    h = jnp.dot(x_ref[...], w1_ref[...], preferred_element_type=jnp.float32)
    g1_ref[...] = h * dinv


def _tc2_body(p0_ref, p1_ref, g1_ref, deg_ref, w2_ref, b1_ref, g2_ref):
    dinv = lax.rsqrt(deg_ref[...])
    s1 = p0_ref[...] + p1_ref[...] + g1_ref[...]
    out1 = jnp.maximum(s1 * dinv + b1_ref[...], 0.0)
    h2 = jnp.dot(out1, w2_ref[...], preferred_element_type=jnp.float32)
    g2_ref[...] = h2 * dinv


def _tc3_body(p0_ref, p1_ref, g2_ref, deg_ref, b2_ref, wfc_ref, bfc_ref, o_ref):
    dinv = lax.rsqrt(deg_ref[...])
    s2 = p0_ref[...] + p1_ref[...] + g2_ref[...]
    out2 = jnp.maximum(s2 * dinv + b2_ref[...], 0.0)
    pooled = jnp.sum(out2, axis=0, keepdims=True) * (1.0 / N_NODES)
    z = jnp.dot(pooled, wfc_ref[...], preferred_element_type=jnp.float32)
    o_ref[...] = jax.nn.sigmoid(z + bfc_ref[...])


def kernel(x, edge_index, W1, b1, W2, b2, Wfc, bfc):
    pad = SLAB_C * CHUNK - EPW
    src3 = jnp.pad(
        edge_index[0].astype(jnp.int32).reshape(NW, EPW), ((0, 0), (0, pad)),
    ).reshape(NW, SLAB_C, CHUNK)
    dst3 = jnp.pad(
        edge_index[1].astype(jnp.int32).reshape(NW, EPW), ((0, 0), (0, pad)),
        constant_values=DUMMY,
    ).reshape(NW, SLAB_C, CHUNK)

    counts = _sc_counts(dst3)
    deg = (counts[0, :N_NODES] + counts[1, :N_NODES] + 1.0)[:, None]

    g1 = pl.pallas_call(
        _tc1_body,
        out_shape=jax.ShapeDtypeStruct((N_NODES, HID), jnp.float32),
    )(deg, x, W1)

    p1 = _sc_scatter_rows(g1, src3, dst3)

    g2 = pl.pallas_call(
        _tc2_body,
        out_shape=jax.ShapeDtypeStruct((N_NODES, HID), jnp.float32),
    )(p1[0, :N_NODES], p1[1, :N_NODES], g1, deg, W2, b1.reshape(1, HID))

    p2 = _sc_scatter_rows(g2, src3, dst3)

    out = pl.pallas_call(
        _tc3_body,
        out_shape=jax.ShapeDtypeStruct((1, 1), jnp.float32),
    )(p2[0, :N_NODES], p2[1, :N_NODES], g2, deg, b2.reshape(1, HID),
      Wfc, bfc.reshape(1, 1))
    return out.reshape(1)


# speedup vs baseline: 1.4792x; 1.4792x over previous
import functools

import jax
import jax.numpy as jnp
from jax import lax
from jax.experimental import pallas as pl
from jax.experimental.pallas import tpu as pltpu
from jax.experimental.pallas import tpu_sc as plsc

N_NODES = 10000
N_EDGES = 320000
IN_DIM = 128
HID = 16

NC = 2
NS = 16
NW = NC * NS

CHUNK = 512
DEPTH = 4
EPW = N_EDGES // NW
CPW = 20
SLAB_C = CPW
E_PAD = NW * SLAB_C * CHUNK

ACC_ROWS = 10240
STRIP = ACC_ROWS // NS
DUMMY = N_NODES

_mesh = plsc.VectorSubcoreMesh(core_axis_name="c", subcore_axis_name="s")


@functools.partial(
    pl.kernel,
    out_type=jax.ShapeDtypeStruct((NC, ACC_ROWS), jnp.float32),
    mesh=_mesh,
    scratch_types=[
        pltpu.VMEM((SLAB_C, CHUNK), jnp.int32),
        pltpu.VMEM((CHUNK,), jnp.float32),
        pltpu.VMEM((STRIP,), jnp.float32),
        pltpu.VMEM_SHARED((ACC_ROWS,), jnp.float32),
    ] + [pltpu.SemaphoreType.DMA] * DEPTH,
    compiler_params=pltpu.CompilerParams(use_tc_tiling_on_sc=False),
)
def _sc_counts(dst_hbm, out_hbm, dst_v, ones_v, stage_v, acc_sh, *sems):
    cid = lax.axis_index("c")
    sid = lax.axis_index("s")
    wid = sid * NC + cid

    ones16 = jnp.ones((16,), jnp.float32)
    for i in range(CHUNK // 16):
        ones_v[pl.ds(i * 16, 16)] = ones16
    zero16 = jnp.zeros((16,), jnp.float32)

    def zero_body(i, carry):
        stage_v[pl.ds(i * 16, 16)] = zero16
        return carry

    lax.fori_loop(0, STRIP // 16, zero_body, 0)
    pltpu.sync_copy(stage_v, acc_sh.at[pl.ds(sid * STRIP, STRIP)])
    pltpu.sync_copy(dst_hbm.at[wid], dst_v)
    plsc.subcore_barrier()

    for b in range(DEPTH):
        pltpu.async_copy(ones_v, acc_sh.at[dst_v.at[b]], sems[b], add=True)

    def body(g, carry):
        for b in range(DEPTH):
            c = g * DEPTH + b
            pltpu.make_async_copy(ones_v, acc_sh.at[dst_v.at[0]], sems[b]).wait()
            pltpu.async_copy(ones_v, acc_sh.at[dst_v.at[c]], sems[b], add=True)
        return carry

    lax.fori_loop(1, CPW // DEPTH, body, 0)
    for b in range(DEPTH):
        pltpu.make_async_copy(ones_v, acc_sh.at[dst_v.at[0]], sems[b]).wait()
    plsc.subcore_barrier()
    pltpu.sync_copy(
        acc_sh.at[pl.ds(sid * STRIP, STRIP)],
        out_hbm.at[cid, pl.ds(sid * STRIP, STRIP)],
    )


@functools.partial(
    pl.kernel,
    out_type=jax.ShapeDtypeStruct((NC, ACC_ROWS, HID), jnp.float32),
    mesh=_mesh,
    scratch_types=[
        pltpu.VMEM((SLAB_C, CHUNK), jnp.int32),
        pltpu.VMEM((SLAB_C, CHUNK), jnp.int32),
    ] + [pltpu.VMEM((CHUNK, HID), jnp.float32)]
      + [pltpu.VMEM_SHARED((ACC_ROWS, HID), jnp.float32)]
      + [pltpu.SemaphoreType.DMA],
    compiler_params=pltpu.CompilerParams(use_tc_tiling_on_sc=False),
)
def _sc_scatter_rows(vals_hbm, src_hbm, dst_hbm, out_hbm,
                     src_v, dst_v, buf, acc_sh, sem):
    cid = lax.axis_index("c")
    sid = lax.axis_index("s")
    wid = sid * NC + cid

    zero16 = jnp.zeros((16,), jnp.float32)

    def zero_buf(i, carry):
        buf[i, :] = zero16
        return carry

    lax.fori_loop(0, CHUNK, zero_buf, 0)
    done = 0
    while done < STRIP:
        step = min(CHUNK, STRIP - done)
        pltpu.sync_copy(
            buf.at[pl.ds(0, step)],
            acc_sh.at[pl.ds(sid * STRIP + done, step)],
        )
        done += step
    pltpu.sync_copy(src_hbm.at[wid], src_v)
    pltpu.sync_copy(dst_hbm.at[wid], dst_v)
    plsc.subcore_barrier()

    def body(j, carry):
        pltpu.async_copy(vals_hbm.at[src_v.at[j]], buf, sem).wait()
        pltpu.sync_copy(buf, acc_sh.at[dst_v.at[j]], add=True)
        return carry

    lax.fori_loop(0, CPW, body, 0)
    plsc.subcore_barrier()
    pltpu.sync_copy(
        acc_sh.at[pl.ds(sid * STRIP, STRIP)],
        out_hbm.at[cid, pl.ds(sid * STRIP, STRIP)],
    )


def _tc1_body(deg_ref, x_ref, w1_ref, g1_ref):
    dinv = lax.rsqrt(deg_ref[...])
    h = jnp.dot(x_ref[...], w1_ref[...], preferred_element_type=jnp.float32)
    g1_ref[...] = h * dinv


def _tc2_body(p0_ref, p1_ref, g1_ref, deg_ref, w2_ref, b1_ref, g2_ref):
    dinv = lax.rsqrt(deg_ref[...])
    s1 = p0_ref[...] + p1_ref[...] + g1_ref[...]
    out1 = jnp.maximum(s1 * dinv + b1_ref[...], 0.0)
    h2 = jnp.dot(out1, w2_ref[...], preferred_element_type=jnp.float32)
    g2_ref[...] = h2 * dinv


def _tc3_body(p0_ref, p1_ref, g2_ref, deg_ref, b2_ref, wfc_ref, bfc_ref, o_ref):
    dinv = lax.rsqrt(deg_ref[...])
    s2 = p0_ref[...] + p1_ref[...] + g2_ref[...]
    out2 = jnp.maximum(s2 * dinv + b2_ref[...], 0.0)
    pooled = jnp.sum(out2, axis=0, keepdims=True) * (1.0 / N_NODES)
    z = jnp.dot(pooled, wfc_ref[...], preferred_element_type=jnp.float32)
    o_ref[...] = jax.nn.sigmoid(z + bfc_ref[...])


def kernel(x, edge_index, W1, b1, W2, b2, Wfc, bfc):
    pad = SLAB_C * CHUNK - EPW
    src3 = jnp.pad(
        edge_index[0].astype(jnp.int32).reshape(NW, EPW), ((0, 0), (0, pad)),
    ).reshape(NW, SLAB_C, CHUNK)
    dst3 = jnp.pad(
        edge_index[1].astype(jnp.int32).reshape(NW, EPW), ((0, 0), (0, pad)),
        constant_values=DUMMY,
    ).reshape(NW, SLAB_C, CHUNK)

    counts = _sc_counts(dst3)
    deg = (counts[0, :N_NODES] + counts[1, :N_NODES] + 1.0)[:, None]

    g1 = pl.pallas_call(
        _tc1_body,
        out_shape=jax.ShapeDtypeStruct((N_NODES, HID), jnp.float32),
    )(deg, x, W1)

    p1 = _sc_scatter_rows(g1, src3, dst3)

    g2 = pl.pallas_call(
        _tc2_body,
        out_shape=jax.ShapeDtypeStruct((N_NODES, HID), jnp.float32),
    )(p1[0, :N_NODES], p1[1, :N_NODES], g1, deg, W2, b1.reshape(1, HID))

    p2 = _sc_scatter_rows(g2, src3, dst3)

    out = pl.pallas_call(
        _tc3_body,
        out_shape=jax.ShapeDtypeStruct((1, 1), jnp.float32),
    )(p2[0, :N_NODES], p2[1, :N_NODES], g2, deg, b2.reshape(1, HID),
      Wfc, bfc.reshape(1, 1))
    return out.reshape(1)


# speedup vs baseline: 1.5106x; 1.0212x over previous
import functools

import jax
import jax.numpy as jnp
from jax import lax
from jax.experimental import pallas as pl
from jax.experimental.pallas import tpu as pltpu
from jax.experimental.pallas import tpu_sc as plsc

N_NODES = 10000
N_EDGES = 320000
IN_DIM = 128
HID = 16

NC = 2
NS = 16
NW = NC * NS

CHUNK = 2048
DEPTH = 1
EPW = N_EDGES // NW
CPW = 5
SLAB_C = CPW
E_PAD = NW * SLAB_C * CHUNK

ACC_ROWS = 10240
STRIP = ACC_ROWS // NS
DUMMY = N_NODES

_mesh = plsc.VectorSubcoreMesh(core_axis_name="c", subcore_axis_name="s")


@functools.partial(
    pl.kernel,
    out_type=jax.ShapeDtypeStruct((NC, ACC_ROWS), jnp.float32),
    mesh=_mesh,
    scratch_types=[
        pltpu.VMEM((SLAB_C, CHUNK), jnp.int32),
        pltpu.VMEM((CHUNK,), jnp.float32),
        pltpu.VMEM((STRIP,), jnp.float32),
        pltpu.VMEM_SHARED((ACC_ROWS,), jnp.float32),
    ] + [pltpu.SemaphoreType.DMA] * DEPTH,
    compiler_params=pltpu.CompilerParams(use_tc_tiling_on_sc=False),
)
def _sc_counts(dst_hbm, out_hbm, dst_v, ones_v, stage_v, acc_sh, *sems):
    cid = lax.axis_index("c")
    sid = lax.axis_index("s")
    wid = sid * NC + cid

    ones16 = jnp.ones((16,), jnp.float32)
    for i in range(CHUNK // 16):
        ones_v[pl.ds(i * 16, 16)] = ones16
    zero16 = jnp.zeros((16,), jnp.float32)

    def zero_body(i, carry):
        stage_v[pl.ds(i * 16, 16)] = zero16
        return carry

    lax.fori_loop(0, STRIP // 16, zero_body, 0)
    pltpu.sync_copy(stage_v, acc_sh.at[pl.ds(sid * STRIP, STRIP)])
    pltpu.sync_copy(dst_hbm.at[wid], dst_v)
    plsc.subcore_barrier()

    for b in range(DEPTH):
        pltpu.async_copy(ones_v, acc_sh.at[dst_v.at[b]], sems[b], add=True)

    def body(g, carry):
        for b in range(DEPTH):
            c = g * DEPTH + b
            pltpu.make_async_copy(ones_v, acc_sh.at[dst_v.at[0]], sems[b]).wait()
            pltpu.async_copy(ones_v, acc_sh.at[dst_v.at[c]], sems[b], add=True)
        return carry

    lax.fori_loop(1, CPW // DEPTH, body, 0)
    for b in range(DEPTH):
        pltpu.make_async_copy(ones_v, acc_sh.at[dst_v.at[0]], sems[b]).wait()
    plsc.subcore_barrier()
    pltpu.sync_copy(
        acc_sh.at[pl.ds(sid * STRIP, STRIP)],
        out_hbm.at[cid, pl.ds(sid * STRIP, STRIP)],
    )


@functools.partial(
    pl.kernel,
    out_type=jax.ShapeDtypeStruct((NC, ACC_ROWS, HID), jnp.float32),
    mesh=_mesh,
    scratch_types=[
        pltpu.VMEM((SLAB_C, CHUNK), jnp.int32),
        pltpu.VMEM((SLAB_C, CHUNK), jnp.int32),
    ] + [pltpu.VMEM((CHUNK, HID), jnp.float32)]
      + [pltpu.VMEM_SHARED((ACC_ROWS, HID), jnp.float32)]
      + [pltpu.SemaphoreType.DMA],
    compiler_params=pltpu.CompilerParams(use_tc_tiling_on_sc=False),
)
def _sc_scatter_rows(vals_hbm, src_hbm, dst_hbm, out_hbm,
                     src_v, dst_v, buf, acc_sh, sem):
    cid = lax.axis_index("c")
    sid = lax.axis_index("s")
    wid = sid * NC + cid

    zero16 = jnp.zeros((16,), jnp.float32)

    def zero_buf(i, carry):
        buf[i, :] = zero16
        return carry

    lax.fori_loop(0, CHUNK, zero_buf, 0)
    done = 0
    while done < STRIP:
        step = min(CHUNK, STRIP - done)
        pltpu.sync_copy(
            buf.at[pl.ds(0, step)],
            acc_sh.at[pl.ds(sid * STRIP + done, step)],
        )
        done += step
    pltpu.sync_copy(src_hbm.at[wid], src_v)
    pltpu.sync_copy(dst_hbm.at[wid], dst_v)
    plsc.subcore_barrier()

    def body(j, carry):
        pltpu.async_copy(vals_hbm.at[src_v.at[j]], buf, sem).wait()
        pltpu.sync_copy(buf, acc_sh.at[dst_v.at[j]], add=True)
        return carry

    lax.fori_loop(0, CPW, body, 0)
    plsc.subcore_barrier()
    pltpu.sync_copy(
        acc_sh.at[pl.ds(sid * STRIP, STRIP)],
        out_hbm.at[cid, pl.ds(sid * STRIP, STRIP)],
    )


def _tc1_body(deg_ref, x_ref, w1_ref, g1_ref):
    dinv = lax.rsqrt(deg_ref[...])
    h = jnp.dot(x_ref[...], w1_ref[...], preferred_element_type=jnp.float32)
    g1_ref[...] = h * dinv


def _tc2_body(p0_ref, p1_ref, g1_ref, deg_ref, w2_ref, b1_ref, g2_ref):
    dinv = lax.rsqrt(deg_ref[...])
    s1 = p0_ref[...] + p1_ref[...] + g1_ref[...]
    out1 = jnp.maximum(s1 * dinv + b1_ref[...], 0.0)
    h2 = jnp.dot(out1, w2_ref[...], preferred_element_type=jnp.float32)
    g2_ref[...] = h2 * dinv


def _tc3_body(p0_ref, p1_ref, g2_ref, deg_ref, b2_ref, wfc_ref, bfc_ref, o_ref):
    dinv = lax.rsqrt(deg_ref[...])
    s2 = p0_ref[...] + p1_ref[...] + g2_ref[...]
    out2 = jnp.maximum(s2 * dinv + b2_ref[...], 0.0)
    pooled = jnp.sum(out2, axis=0, keepdims=True) * (1.0 / N_NODES)
    z = jnp.dot(pooled, wfc_ref[...], preferred_element_type=jnp.float32)
    o_ref[...] = jax.nn.sigmoid(z + bfc_ref[...])


def kernel(x, edge_index, W1, b1, W2, b2, Wfc, bfc):
    pad = SLAB_C * CHUNK - EPW
    src3 = jnp.pad(
        edge_index[0].astype(jnp.int32).reshape(NW, EPW), ((0, 0), (0, pad)),
    ).reshape(NW, SLAB_C, CHUNK)
    dst3 = jnp.pad(
        edge_index[1].astype(jnp.int32).reshape(NW, EPW), ((0, 0), (0, pad)),
        constant_values=DUMMY,
    ).reshape(NW, SLAB_C, CHUNK)

    counts = _sc_counts(dst3)
    deg = (counts[0, :N_NODES] + counts[1, :N_NODES] + 1.0)[:, None]

    g1 = pl.pallas_call(
        _tc1_body,
        out_shape=jax.ShapeDtypeStruct((N_NODES, HID), jnp.float32),
    )(deg, x, W1)

    p1 = _sc_scatter_rows(g1, src3, dst3)

    g2 = pl.pallas_call(
        _tc2_body,
        out_shape=jax.ShapeDtypeStruct((N_NODES, HID), jnp.float32),
    )(p1[0, :N_NODES], p1[1, :N_NODES], g1, deg, W2, b1.reshape(1, HID))

    p2 = _sc_scatter_rows(g2, src3, dst3)

    out = pl.pallas_call(
        _tc3_body,
        out_shape=jax.ShapeDtypeStruct((1, 1), jnp.float32),
    )(p2[0, :N_NODES], p2[1, :N_NODES], g2, deg, b2.reshape(1, HID),
      Wfc, bfc.reshape(1, 1))
    return out.reshape(1)


# speedup vs baseline: 1.6753x; 1.1090x over previous
import functools

import jax
import jax.numpy as jnp
from jax import lax
from jax.experimental import pallas as pl
from jax.experimental.pallas import tpu as pltpu
from jax.experimental.pallas import tpu_sc as plsc

N_NODES = 10000
N_EDGES = 320000
IN_DIM = 128
HID = 16

NC = 2
NS = 16
NW = NC * NS

CHUNK = 1024
EPW = N_EDGES // NW
CPW = 10
NBUF = 4
SLAB_C = CPW
E_PAD = NW * SLAB_C * CHUNK

ACC_ROWS = 10240
STRIP = ACC_ROWS // NS
DUMMY = N_NODES

_mesh = plsc.VectorSubcoreMesh(core_axis_name="c", subcore_axis_name="s")


@functools.partial(
    pl.kernel,
    out_type=jax.ShapeDtypeStruct((NC, ACC_ROWS), jnp.float32),
    mesh=_mesh,
    scratch_types=[
        pltpu.VMEM((SLAB_C, CHUNK), jnp.int32),
        pltpu.VMEM((CHUNK,), jnp.float32),
        pltpu.VMEM((STRIP,), jnp.float32),
        pltpu.VMEM_SHARED((ACC_ROWS,), jnp.float32),
    ] + [pltpu.SemaphoreType.DMA] * CPW,
    compiler_params=pltpu.CompilerParams(use_tc_tiling_on_sc=False),
)
def _sc_counts(dst_hbm, out_hbm, dst_v, ones_v, stage_v, acc_sh, *sems):
    cid = lax.axis_index("c")
    sid = lax.axis_index("s")
    wid = sid * NC + cid

    ones16 = jnp.ones((16,), jnp.float32)
    for i in range(CHUNK // 16):
        ones_v[pl.ds(i * 16, 16)] = ones16
    zero16 = jnp.zeros((16,), jnp.float32)

    def zero_body(i, carry):
        stage_v[pl.ds(i * 16, 16)] = zero16
        return carry

    lax.fori_loop(0, STRIP // 16, zero_body, 0)
    pltpu.sync_copy(stage_v, acc_sh.at[pl.ds(sid * STRIP, STRIP)])
    pltpu.sync_copy(dst_hbm.at[wid], dst_v)
    plsc.subcore_barrier()

    copies = [
        pltpu.async_copy(ones_v, acc_sh.at[dst_v.at[c]], sems[c], add=True)
        for c in range(CPW)
    ]
    for cp in copies:
        cp.wait()
    plsc.subcore_barrier()
    pltpu.sync_copy(
        acc_sh.at[pl.ds(sid * STRIP, STRIP)],
        out_hbm.at[cid, pl.ds(sid * STRIP, STRIP)],
    )


@functools.partial(
    pl.kernel,
    out_type=jax.ShapeDtypeStruct((NC, ACC_ROWS, HID), jnp.float32),
    mesh=_mesh,
    scratch_types=[
        pltpu.VMEM((SLAB_C, CHUNK), jnp.int32),
        pltpu.VMEM((SLAB_C, CHUNK), jnp.int32),
    ] + [pltpu.VMEM((CHUNK, HID), jnp.float32)] * NBUF
      + [pltpu.VMEM_SHARED((ACC_ROWS, HID), jnp.float32)]
      + [pltpu.SemaphoreType.DMA] * (2 * CPW),
    compiler_params=pltpu.CompilerParams(use_tc_tiling_on_sc=False),
)
def _sc_scatter_rows(vals_hbm, src_hbm, dst_hbm, out_hbm,
                     src_v, dst_v, *rest):
    bufs = rest[:NBUF]
    acc_sh = rest[NBUF]
    semg = rest[NBUF + 1:NBUF + 1 + CPW]
    sems = rest[NBUF + 1 + CPW:]
    cid = lax.axis_index("c")
    sid = lax.axis_index("s")
    wid = sid * NC + cid

    zero16 = jnp.zeros((16,), jnp.float32)

    def zero_buf(i, carry):
        bufs[0][i, :] = zero16
        return carry

    lax.fori_loop(0, CHUNK, zero_buf, 0)
    done = 0
    while done < STRIP:
        step = min(CHUNK, STRIP - done)
        pltpu.sync_copy(
            bufs[0].at[pl.ds(0, step)],
            acc_sh.at[pl.ds(sid * STRIP + done, step)],
        )
        done += step
    pltpu.sync_copy(src_hbm.at[wid], src_v)
    pltpu.sync_copy(dst_hbm.at[wid], dst_v)
    plsc.subcore_barrier()

    LEAD = NBUF - 2
    gathers = [None] * CPW
    scatters = [None] * CPW
    for c in range(LEAD):
        gathers[c] = pltpu.async_copy(
            vals_hbm.at[src_v.at[c]], bufs[c % NBUF], semg[c])
    for c in range(CPW):
        gathers[c].wait()
        scatters[c] = pltpu.async_copy(
            bufs[c % NBUF], acc_sh.at[dst_v.at[c]], sems[c], add=True)
        j = c + LEAD
        if j < CPW:
            if j - NBUF >= 0:
                scatters[j - NBUF].wait()
            gathers[j] = pltpu.async_copy(
                vals_hbm.at[src_v.at[j]], bufs[j % NBUF], semg[j])
    for c in range(CPW - NBUF, CPW):
        scatters[c].wait()
    plsc.subcore_barrier()
    pltpu.sync_copy(
        acc_sh.at[pl.ds(sid * STRIP, STRIP)],
        out_hbm.at[cid, pl.ds(sid * STRIP, STRIP)],
    )


def _tc1_body(deg_ref, x_ref, w1_ref, g1_ref):
    dinv = lax.rsqrt(deg_ref[...])
    h = jnp.dot(x_ref[...], w1_ref[...], preferred_element_type=jnp.float32)
    g1_ref[...] = h * dinv


def _tc2_body(p0_ref, p1_ref, g1_ref, deg_ref, w2_ref, b1_ref, g2_ref):
    dinv = lax.rsqrt(deg_ref[...])
    s1 = p0_ref[...] + p1_ref[...] + g1_ref[...]
    out1 = jnp.maximum(s1 * dinv + b1_ref[...], 0.0)
    h2 = jnp.dot(out1, w2_ref[...], preferred_element_type=jnp.float32)
    g2_ref[...] = h2 * dinv


def _tc3_body(p0_ref, p1_ref, g2_ref, deg_ref, b2_ref, wfc_ref, bfc_ref, o_ref):
    dinv = lax.rsqrt(deg_ref[...])
    s2 = p0_ref[...] + p1_ref[...] + g2_ref[...]
    out2 = jnp.maximum(s2 * dinv + b2_ref[...], 0.0)
    pooled = jnp.sum(out2, axis=0, keepdims=True) * (1.0 / N_NODES)
    z = jnp.dot(pooled, wfc_ref[...], preferred_element_type=jnp.float32)
    o_ref[...] = jax.nn.sigmoid(z + bfc_ref[...])


def kernel(x, edge_index, W1, b1, W2, b2, Wfc, bfc):
    pad = SLAB_C * CHUNK - EPW
    src3 = jnp.pad(
        edge_index[0].astype(jnp.int32).reshape(NW, EPW), ((0, 0), (0, pad)),
    ).reshape(NW, SLAB_C, CHUNK)
    dst3 = jnp.pad(
        edge_index[1].astype(jnp.int32).reshape(NW, EPW), ((0, 0), (0, pad)),
        constant_values=DUMMY,
    ).reshape(NW, SLAB_C, CHUNK)

    counts = _sc_counts(dst3)
    deg = (counts[0, :N_NODES] + counts[1, :N_NODES] + 1.0)[:, None]

    g1 = pl.pallas_call(
        _tc1_body,
        out_shape=jax.ShapeDtypeStruct((N_NODES, HID), jnp.float32),
    )(deg, x, W1)

    p1 = _sc_scatter_rows(g1, src3, dst3)

    g2 = pl.pallas_call(
        _tc2_body,
        out_shape=jax.ShapeDtypeStruct((N_NODES, HID), jnp.float32),
    )(p1[0, :N_NODES], p1[1, :N_NODES], g1, deg, W2, b1.reshape(1, HID))

    p2 = _sc_scatter_rows(g2, src3, dst3)

    out = pl.pallas_call(
        _tc3_body,
        out_shape=jax.ShapeDtypeStruct((1, 1), jnp.float32),
    )(p2[0, :N_NODES], p2[1, :N_NODES], g2, deg, b2.reshape(1, HID),
      Wfc, bfc.reshape(1, 1))
    return out.reshape(1)


# speedup vs baseline: 2.2968x; 1.3710x over previous
import functools

import jax
import jax.numpy as jnp
from jax import lax
from jax.experimental import pallas as pl
from jax.experimental.pallas import tpu as pltpu
from jax.experimental.pallas import tpu_sc as plsc

N_NODES = 10000
N_EDGES = 320000
IN_DIM = 128
HID = 16

NC = 2
NS = 16
NW = NC * NS

CHUNK = 1024
EPW = N_EDGES // NW
CPW = 10
NBUF = 4
SLAB_C = CPW
E_PAD = NW * SLAB_C * CHUNK

ACC_ROWS = 10240
STRIP = ACC_ROWS // NS
DUMMY = N_NODES

_mesh = plsc.VectorSubcoreMesh(core_axis_name="c", subcore_axis_name="s")


@functools.partial(
    pl.kernel,
    out_type=jax.ShapeDtypeStruct((NC, ACC_ROWS), jnp.float32),
    mesh=_mesh,
    scratch_types=[
        pltpu.VMEM((SLAB_C, CHUNK), jnp.int32),
        pltpu.VMEM((CHUNK,), jnp.float32),
        pltpu.VMEM((STRIP,), jnp.float32),
        pltpu.VMEM_SHARED((ACC_ROWS,), jnp.float32),
    ] + [pltpu.SemaphoreType.DMA] * CPW,
    compiler_params=pltpu.CompilerParams(use_tc_tiling_on_sc=False),
)
def _sc_counts(dst_hbm, out_hbm, dst_v, ones_v, stage_v, acc_sh, *sems):
    cid = lax.axis_index("c")
    sid = lax.axis_index("s")
    wid = sid * NC + cid

    ones16 = jnp.ones((16,), jnp.float32)
    for i in range(CHUNK // 16):
        ones_v[pl.ds(i * 16, 16)] = ones16
    zero16 = jnp.zeros((16,), jnp.float32)

    def zero_body(i, carry):
        stage_v[pl.ds(i * 16, 16)] = zero16
        return carry

    lax.fori_loop(0, STRIP // 16, zero_body, 0)
    pltpu.sync_copy(stage_v, acc_sh.at[pl.ds(sid * STRIP, STRIP)])
    pltpu.sync_copy(dst_hbm.at[wid], dst_v)
    plsc.subcore_barrier()

    copies = [
        pltpu.async_copy(ones_v, acc_sh.at[dst_v.at[c]], sems[c], add=True)
        for c in range(CPW)
    ]
    for cp in copies:
        cp.wait()
    plsc.subcore_barrier()
    pltpu.sync_copy(
        acc_sh.at[pl.ds(sid * STRIP, STRIP)],
        out_hbm.at[cid, pl.ds(sid * STRIP, STRIP)],
    )


@functools.partial(
    pl.kernel,
    out_type=jax.ShapeDtypeStruct((NC, ACC_ROWS, HID), jnp.float32),
    mesh=_mesh,
    scratch_types=[
        pltpu.VMEM((SLAB_C, CHUNK), jnp.int32),
        pltpu.VMEM((SLAB_C, CHUNK), jnp.int32),
    ] + [pltpu.VMEM((CHUNK, HID), jnp.float32)] * NBUF
      + [pltpu.VMEM_SHARED((ACC_ROWS, HID), jnp.float32)]
      + [pltpu.VMEM_SHARED((ACC_ROWS, HID), jnp.float32)]
      + [pltpu.SemaphoreType.DMA] * (2 * CPW),
    compiler_params=pltpu.CompilerParams(use_tc_tiling_on_sc=False),
)
def _sc_scatter_rows(vals_hbm, src_hbm, dst_hbm, out_hbm,
                     src_v, dst_v, *rest):
    bufs = rest[:NBUF]
    acc_sh = rest[NBUF]
    table_sh = rest[NBUF + 1]
    semg = rest[NBUF + 2:NBUF + 2 + CPW]
    sems = rest[NBUF + 2 + CPW:]
    cid = lax.axis_index("c")
    sid = lax.axis_index("s")
    wid = sid * NC + cid

    pltpu.sync_copy(
        vals_hbm.at[pl.ds(sid * STRIP, STRIP)],
        table_sh.at[pl.ds(sid * STRIP, STRIP)],
    )
    zero16 = jnp.zeros((16,), jnp.float32)

    def zero_buf(i, carry):
        bufs[0][i, :] = zero16
        return carry

    lax.fori_loop(0, CHUNK, zero_buf, 0)
    done = 0
    while done < STRIP:
        step = min(CHUNK, STRIP - done)
        pltpu.sync_copy(
            bufs[0].at[pl.ds(0, step)],
            acc_sh.at[pl.ds(sid * STRIP + done, step)],
        )
        done += step
    pltpu.sync_copy(src_hbm.at[wid], src_v)
    pltpu.sync_copy(dst_hbm.at[wid], dst_v)
    plsc.subcore_barrier()

    LEAD = NBUF - 2
    gathers = [None] * CPW
    scatters = [None] * CPW
    for c in range(LEAD):
        gathers[c] = pltpu.async_copy(
            table_sh.at[src_v.at[c]], bufs[c % NBUF], semg[c])
    for c in range(CPW):
        gathers[c].wait()
        scatters[c] = pltpu.async_copy(
            bufs[c % NBUF], acc_sh.at[dst_v.at[c]], sems[c], add=True)
        j = c + LEAD
        if j < CPW:
            if j - NBUF >= 0:
                scatters[j - NBUF].wait()
            gathers[j] = pltpu.async_copy(
                table_sh.at[src_v.at[j]], bufs[j % NBUF], semg[j])
    for c in range(CPW - NBUF, CPW):
        scatters[c].wait()
    plsc.subcore_barrier()
    pltpu.sync_copy(
        acc_sh.at[pl.ds(sid * STRIP, STRIP)],
        out_hbm.at[cid, pl.ds(sid * STRIP, STRIP)],
    )


def _tc1_body(deg_ref, x_ref, w1_ref, g1_ref):
    dinv = lax.rsqrt(deg_ref[...])
    h = jnp.dot(x_ref[...], w1_ref[...], preferred_element_type=jnp.float32)
    g1_ref[...] = h * dinv


def _tc2_body(p0_ref, p1_ref, g1_ref, deg_ref, w2_ref, b1_ref, g2_ref):
    dinv = lax.rsqrt(deg_ref[...])
    s1 = p0_ref[...] + p1_ref[...] + g1_ref[...]
    out1 = jnp.maximum(s1 * dinv + b1_ref[...], 0.0)
    h2 = jnp.dot(out1, w2_ref[...], preferred_element_type=jnp.float32)
    g2_ref[...] = h2 * dinv


def _tc3_body(p0_ref, p1_ref, g2_ref, deg_ref, b2_ref, wfc_ref, bfc_ref, o_ref):
    dinv = lax.rsqrt(deg_ref[...])
    s2 = p0_ref[...] + p1_ref[...] + g2_ref[...]
    out2 = jnp.maximum(s2 * dinv + b2_ref[...], 0.0)
    pooled = jnp.sum(out2, axis=0, keepdims=True) * (1.0 / N_NODES)
    z = jnp.dot(pooled, wfc_ref[...], preferred_element_type=jnp.float32)
    o_ref[...] = jax.nn.sigmoid(z + bfc_ref[...])


def kernel(x, edge_index, W1, b1, W2, b2, Wfc, bfc):
    pad = SLAB_C * CHUNK - EPW
    src3 = jnp.pad(
        edge_index[0].astype(jnp.int32).reshape(NW, EPW), ((0, 0), (0, pad)),
    ).reshape(NW, SLAB_C, CHUNK)
    dst3 = jnp.pad(
        edge_index[1].astype(jnp.int32).reshape(NW, EPW), ((0, 0), (0, pad)),
        constant_values=DUMMY,
    ).reshape(NW, SLAB_C, CHUNK)

    counts = _sc_counts(dst3)
    deg = (counts[0, :N_NODES] + counts[1, :N_NODES] + 1.0)[:, None]

    g1 = pl.pallas_call(
        _tc1_body,
        out_shape=jax.ShapeDtypeStruct((N_NODES, HID), jnp.float32),
    )(deg, x, W1)

    p1 = _sc_scatter_rows(
        jnp.pad(g1, ((0, ACC_ROWS - N_NODES), (0, 0))), src3, dst3)

    g2 = pl.pallas_call(
        _tc2_body,
        out_shape=jax.ShapeDtypeStruct((N_NODES, HID), jnp.float32),
    )(p1[0, :N_NODES], p1[1, :N_NODES], g1, deg, W2, b1.reshape(1, HID))

    p2 = _sc_scatter_rows(
        jnp.pad(g2, ((0, ACC_ROWS - N_NODES), (0, 0))), src3, dst3)

    out = pl.pallas_call(
        _tc3_body,
        out_shape=jax.ShapeDtypeStruct((1, 1), jnp.float32),
    )(p2[0, :N_NODES], p2[1, :N_NODES], g2, deg, b2.reshape(1, HID),
      Wfc, bfc.reshape(1, 1))
    return out.reshape(1)
